# baseline (speedup 1.0000x reference)
import functools

import jax
from jax import lax
import jax.numpy as jnp
from jax.experimental import pallas as pl
from jax.experimental.pallas import tpu as pltpu
from jax.experimental.pallas import tpu_sc as plsc

_NUM_EXPERTS = 8
_TOP_K = 2
_AUX_COEF = 0.02
_IGNORE = -100

_R = 128
_V = 32000
_N = 2048
_L = 32
_E = 8

_NW = 32
_PW = _N // _NW
_SCL = 16

_VTC = 16000
_VSC = _V - _VTC
_UF = 8


_LPB = 2


def _fused_body(x_ref, r_ref, lab2_ref,
                m_ref, s_ref, cp_ref, den_ref,
                cnte_ref, pe_ref, ws_ref):
    i = pl.program_id(0)
    ni = pl.num_programs(0)

    @pl.when(i == 0)
    def _init_out():
        cnte_ref[:, :] = jnp.zeros((_E, 128), jnp.float32)
        pe_ref[:, :] = jnp.zeros((_E, 128), jnp.float32)
        ws_ref[:, :] = jnp.zeros((1, 128), jnp.float32)

    x = x_ref[:, :].reshape(_R, _VTC // 128, 128)
    m = jnp.max(x, axis=1)
    s = jnp.sum(jnp.exp(x - m[:, None, :]), axis=1)
    rm = jnp.max(m, axis=1, keepdims=True)
    s2 = jnp.sum(s * jnp.exp(m - rm), axis=1, keepdims=True)
    m_ref[:, :] = rm
    s_ref[:, :] = s2

    w = (lab2_ref[:, :] != _IGNORE).astype(jnp.float32)
    sub = jax.lax.broadcasted_iota(jnp.int32, (_E, _N), 0)
    for li in range(_LPB):
        xr = r_ref[li]
        mx = jnp.max(xr, axis=0, keepdims=True)
        ex = jnp.exp(xr - mx)
        prob = ex / jnp.sum(ex, axis=0, keepdims=True)

        rank = jnp.zeros((_E, _N), jnp.float32)
        for j in range(_E):
            xj = xr[j:j + 1, :]
            beats = jnp.logical_or(
                xj > xr, jnp.logical_and(xj == xr, j < sub))
            rank += beats.astype(jnp.float32)
        ind = (rank < _TOP_K).astype(jnp.float32)

        cnte_ref[:, :] += jnp.sum(
            (ind * w).reshape(_E, _N // 128, 128), axis=1)
        pe_ref[:, :] += jnp.sum(
            (prob * w).reshape(_E, _N // 128, 128), axis=1)
        ws_ref[:, :] += jnp.sum(w.reshape(1, _N // 128, 128), axis=1)

    @pl.when(i == ni - 1)
    def _finish():
        cnt = jnp.sum(cnte_ref[:, :], axis=1, keepdims=True)
        pe = jnp.sum(pe_ref[:, :], axis=1, keepdims=True)
        cp_ref[:, :] = jnp.sum(cnt * pe).reshape(1, 1)
        den_ref[:, :] = jnp.sum(ws_ref[:, :]).reshape(1, 1)


def _sc_body(logits_hbm, lab_hbm, g_out, m_out, s_out,
             lab_v, idx_v, g_v, acc_v,
             buf0, buf1, m_buf, s_buf, sem, sem0, sem1):
    c = lax.axis_index("c")
    s = lax.axis_index("s")
    wid = s * 2 + c
    base = wid * _PW

    pltpu.sync_copy(lab_hbm.at[pl.ds(base, _PW)], lab_v)
    for ch in range(_PW // _SCL):
        lab = lab_v[pl.ds(ch * _SCL, _SCL)]
        pos = base + ch * _SCL + lax.iota(jnp.int32, _SCL)
        idx_v[pl.ds(ch * _SCL, _SCL)] = jnp.where(lab >= 0, pos * _V + lab, 0)
    pltpu.async_copy(logits_hbm.at[idx_v], g_v, sem).wait()
    acc = jnp.zeros((_SCL,), jnp.float32)
    for ch in range(_PW // _SCL):
        lab = lab_v[pl.ds(ch * _SCL, _SCL)]
        g = g_v[pl.ds(ch * _SCL, _SCL)]
        acc = acc + jnp.where(lab >= 0, g, 0.0)
    acc_v[...] = acc
    pltpu.sync_copy(acc_v, g_out.at[wid])

    def _start(r, buf, dsem):
        off = (base + r) * _V + _VTC
        pltpu.async_copy(logits_hbm.at[pl.ds(off, _VSC)], buf, dsem)

    def _wait(buf, dsem):
        pltpu.make_async_copy(logits_hbm.at[pl.ds(0, _VSC)], buf, dsem).wait()

    def _row_lse(buf, r):
        def mx_body(k, carry):
            a, b = carry
            for u in range(_UF):
                v = buf[pl.ds(k * _UF * _SCL + u * _SCL, _SCL)]
                if u % 2 == 0:
                    a = jnp.maximum(a, v)
                else:
                    b = jnp.maximum(b, v)
            return a, b
        init = jnp.full((_SCL,), -jnp.inf, jnp.float32)
        ma, mb = lax.fori_loop(0, _VSC // (_UF * _SCL), mx_body, (init, init))
        m16 = jnp.maximum(ma, mb)

        def sm_body(k, carry):
            a, b = carry
            for u in range(_UF):
                v = buf[pl.ds(k * _UF * _SCL + u * _SCL, _SCL)]
                e = jnp.exp(v - m16)
                if u % 2 == 0:
                    a = a + e
                else:
                    b = b + e
            return a, b
        z = jnp.zeros((_SCL,), jnp.float32)
        sa, sb = lax.fori_loop(0, _VSC // (_UF * _SCL), sm_body, (z, z))
        m_buf[pl.ds(r * _SCL, _SCL)] = m16
        s_buf[pl.ds(r * _SCL, _SCL)] = sa + sb

    _start(0, buf0, sem0)

    def row_pair(gidx, carry):
        r0 = 2 * gidx
        _start(r0 + 1, buf1, sem1)
        _wait(buf0, sem0)
        _row_lse(buf0, r0)

        @pl.when(gidx < _PW // 2 - 1)
        def _prefetch():
            _start(r0 + 2, buf0, sem0)

        _wait(buf1, sem1)
        _row_lse(buf1, r0 + 1)
        return carry

    lax.fori_loop(0, _PW // 2, row_pair, 0)
    pltpu.sync_copy(m_buf, m_out.at[wid])
    pltpu.sync_copy(s_buf, s_out.at[wid])


def _combine_body(mt_ref, st_ref, ms_ref, ss_ref, lab_ref, g_ref,
                  cp_ref, den_ref, out_ref):
    mt = mt_ref[:, :]
    st = st_ref[:, :]
    m3 = ms_ref[...]
    s3 = ss_ref[...]
    ms = jnp.max(m3, axis=2)
    ss = jnp.sum(s3 * jnp.exp(m3 - ms[:, :, None]), axis=2)
    big = jnp.maximum(mt, ms)
    tot = st * jnp.exp(mt - big) + ss * jnp.exp(ms - big)
    lse = big + jnp.log(tot)
    valid = lab_ref[:, :] != _IGNORE
    num = jnp.sum(jnp.where(valid, lse, 0.0)) - jnp.sum(g_ref[:, :])
    cnt = jnp.sum(valid.astype(jnp.float32))
    aux = _NUM_EXPERTS * cp_ref[0, 0] / (den_ref[0, 0] * den_ref[0, 0])
    out_ref[:, :] = (num / cnt + _AUX_COEF * aux).reshape(1, 1)


def kernel(logits, labels, router_logits):
    n = logits.shape[1]
    v = logits.shape[-1]
    logits2 = logits.reshape(n, v)
    lab_flat = labels.reshape(-1)
    shift_lab = jnp.concatenate(
        [lab_flat[1:], jnp.full((1,), _IGNORE, jnp.int32)])

    sc_call = functools.partial(
        pl.kernel,
        mesh=plsc.VectorSubcoreMesh(core_axis_name="c", subcore_axis_name="s"),
        out_type=[
            jax.ShapeDtypeStruct((_NW, _SCL), jnp.float32),
            jax.ShapeDtypeStruct((_NW, _PW * _SCL), jnp.float32),
            jax.ShapeDtypeStruct((_NW, _PW * _SCL), jnp.float32),
        ],
        scratch_types=[
            pltpu.VMEM((_PW,), jnp.int32),
            pltpu.VMEM((_PW,), jnp.int32),
            pltpu.VMEM((_PW,), jnp.float32),
            pltpu.VMEM((_SCL,), jnp.float32),
            pltpu.VMEM((_VSC,), jnp.float32),
            pltpu.VMEM((_VSC,), jnp.float32),
            pltpu.VMEM((_PW * _SCL,), jnp.float32),
            pltpu.VMEM((_PW * _SCL,), jnp.float32),
            pltpu.SemaphoreType.DMA,
            pltpu.SemaphoreType.DMA,
            pltpu.SemaphoreType.DMA,
        ],
    )(_sc_body)
    g_partials, m_sc, s_sc = sc_call(logits.reshape(-1), shift_lab)

    lab2 = lab_flat.reshape(1, n)
    router_t = router_logits.transpose(0, 2, 1)
    m_tc, s_tc, cp, den = pl.pallas_call(
        _fused_body,
        grid=(n // _R,),
        in_specs=[
            pl.BlockSpec((_R, _VTC), lambda i: (i, 0)),
            pl.BlockSpec((_LPB, _E, _N), lambda i: (i, 0, 0)),
            pl.BlockSpec((1, _N), lambda i: (0, 0)),
        ],
        out_specs=[
            pl.BlockSpec((_R, 1), lambda i: (i, 0)),
            pl.BlockSpec((_R, 1), lambda i: (i, 0)),
            pl.BlockSpec((1, 1), lambda i: (0, 0)),
            pl.BlockSpec((1, 1), lambda i: (0, 0)),
        ],
        out_shape=[
            jax.ShapeDtypeStruct((n, 1), jnp.float32),
            jax.ShapeDtypeStruct((n, 1), jnp.float32),
            jax.ShapeDtypeStruct((1, 1), jnp.float32),
            jax.ShapeDtypeStruct((1, 1), jnp.float32),
        ],
        scratch_shapes=[
            pltpu.VMEM((_E, 128), jnp.float32),
            pltpu.VMEM((_E, 128), jnp.float32),
            pltpu.VMEM((1, 128), jnp.float32),
        ],
        compiler_params=pltpu.CompilerParams(
            dimension_semantics=("arbitrary",)),
    )(logits2, router_t, lab2)

    shp = (n // 128, 128)
    shp3 = (n // 128, 128, _SCL)
    loss = pl.pallas_call(
        _combine_body,
        grid=(1,),
        in_specs=[
            pl.BlockSpec(shp, lambda i: (0, 0)),
            pl.BlockSpec(shp, lambda i: (0, 0)),
            pl.BlockSpec(shp3, lambda i: (0, 0, 0)),
            pl.BlockSpec(shp3, lambda i: (0, 0, 0)),
            pl.BlockSpec(shp, lambda i: (0, 0)),
            pl.BlockSpec((_NW, _SCL), lambda i: (0, 0)),
            pl.BlockSpec((1, 1), lambda i: (0, 0)),
            pl.BlockSpec((1, 1), lambda i: (0, 0)),
        ],
        out_specs=pl.BlockSpec((1, 1), lambda i: (0, 0)),
        out_shape=jax.ShapeDtypeStruct((1, 1), jnp.float32),
    )(m_tc.reshape(shp), s_tc.reshape(shp),
      m_sc.reshape(shp3), s_sc.reshape(shp3),
      shift_lab.reshape(shp), g_partials, cp, den)
    return loss[0, 0]


# speedup vs baseline: 1.0234x; 1.0234x over previous
import functools

import jax
from jax import lax
import jax.numpy as jnp
from jax.experimental import pallas as pl
from jax.experimental.pallas import tpu as pltpu
from jax.experimental.pallas import tpu_sc as plsc

_NUM_EXPERTS = 8
_TOP_K = 2
_AUX_COEF = 0.02
_IGNORE = -100

_R = 128
_V = 32000
_N = 2048
_L = 32
_E = 8

_NW = 32
_PW = _N // _NW
_SCL = 16

_VTC = 22400
_VSC = _V - _VTC
_UF = 8


_LPB = 2


def _fused_body(x_ref, r_ref, lab2_ref,
                m_ref, s_ref, cp_ref, den_ref,
                cnte_ref, pe_ref, ws_ref):
    i = pl.program_id(0)
    ni = pl.num_programs(0)

    @pl.when(i == 0)
    def _init_out():
        cnte_ref[:, :] = jnp.zeros((_E, 128), jnp.float32)
        pe_ref[:, :] = jnp.zeros((_E, 128), jnp.float32)
        ws_ref[:, :] = jnp.zeros((1, 128), jnp.float32)

    x = x_ref[:, :].reshape(_R, _VTC // 128, 128)
    m = jnp.max(x, axis=1)
    s = jnp.sum(jnp.exp(x - m[:, None, :]), axis=1)
    rm = jnp.max(m, axis=1, keepdims=True)
    s2 = jnp.sum(s * jnp.exp(m - rm), axis=1, keepdims=True)
    m_ref[:, :] = rm
    s_ref[:, :] = s2

    w = (lab2_ref[:, :] != _IGNORE).astype(jnp.float32)
    sub = jax.lax.broadcasted_iota(jnp.int32, (_E, _N), 0)
    for li in range(_LPB):
        xr = r_ref[li]
        mx = jnp.max(xr, axis=0, keepdims=True)
        ex = jnp.exp(xr - mx)
        prob = ex / jnp.sum(ex, axis=0, keepdims=True)

        rank = jnp.zeros((_E, _N), jnp.float32)
        for j in range(_E):
            xj = xr[j:j + 1, :]
            beats = jnp.logical_or(
                xj > xr, jnp.logical_and(xj == xr, j < sub))
            rank += beats.astype(jnp.float32)
        ind = (rank < _TOP_K).astype(jnp.float32)

        cnte_ref[:, :] += jnp.sum(
            (ind * w).reshape(_E, _N // 128, 128), axis=1)
        pe_ref[:, :] += jnp.sum(
            (prob * w).reshape(_E, _N // 128, 128), axis=1)
        ws_ref[:, :] += jnp.sum(w.reshape(1, _N // 128, 128), axis=1)

    @pl.when(i == ni - 1)
    def _finish():
        cnt = jnp.sum(cnte_ref[:, :], axis=1, keepdims=True)
        pe = jnp.sum(pe_ref[:, :], axis=1, keepdims=True)
        cp_ref[:, :] = jnp.sum(cnt * pe).reshape(1, 1)
        den_ref[:, :] = jnp.sum(ws_ref[:, :]).reshape(1, 1)


def _sc_body(logits_hbm, lab_hbm, g_out, m_out, s_out,
             lab_v, idx_v, g_v, acc_v,
             buf0, buf1, m_buf, s_buf, sem, sem0, sem1):
    c = lax.axis_index("c")
    s = lax.axis_index("s")
    wid = s * 2 + c
    base = wid * _PW

    pltpu.sync_copy(lab_hbm.at[pl.ds(base, _PW)], lab_v)
    for ch in range(_PW // _SCL):
        lab = lab_v[pl.ds(ch * _SCL, _SCL)]
        pos = base + ch * _SCL + lax.iota(jnp.int32, _SCL)
        idx_v[pl.ds(ch * _SCL, _SCL)] = jnp.where(lab >= 0, pos * _V + lab, 0)
    pltpu.async_copy(logits_hbm.at[idx_v], g_v, sem).wait()
    acc = jnp.zeros((_SCL,), jnp.float32)
    for ch in range(_PW // _SCL):
        lab = lab_v[pl.ds(ch * _SCL, _SCL)]
        g = g_v[pl.ds(ch * _SCL, _SCL)]
        acc = acc + jnp.where(lab >= 0, g, 0.0)
    acc_v[...] = acc
    pltpu.sync_copy(acc_v, g_out.at[wid])

    def _start(r, buf, dsem):
        off = (base + r) * _V + _VTC
        pltpu.async_copy(logits_hbm.at[pl.ds(off, _VSC)], buf, dsem)

    def _wait(buf, dsem):
        pltpu.make_async_copy(logits_hbm.at[pl.ds(0, _VSC)], buf, dsem).wait()

    def _row_lse(buf, r):
        def mx_body(k, carry):
            a, b = carry
            for u in range(_UF):
                v = buf[pl.ds(k * _UF * _SCL + u * _SCL, _SCL)]
                if u % 2 == 0:
                    a = jnp.maximum(a, v)
                else:
                    b = jnp.maximum(b, v)
            return a, b
        init = jnp.full((_SCL,), -jnp.inf, jnp.float32)
        ma, mb = lax.fori_loop(0, _VSC // (_UF * _SCL), mx_body, (init, init))
        m16 = jnp.maximum(ma, mb)

        def sm_body(k, carry):
            a, b = carry
            for u in range(_UF):
                v = buf[pl.ds(k * _UF * _SCL + u * _SCL, _SCL)]
                e = jnp.exp(v - m16)
                if u % 2 == 0:
                    a = a + e
                else:
                    b = b + e
            return a, b
        z = jnp.zeros((_SCL,), jnp.float32)
        sa, sb = lax.fori_loop(0, _VSC // (_UF * _SCL), sm_body, (z, z))
        m_buf[pl.ds(r * _SCL, _SCL)] = m16
        s_buf[pl.ds(r * _SCL, _SCL)] = sa + sb

    _start(0, buf0, sem0)

    def row_pair(gidx, carry):
        r0 = 2 * gidx
        _start(r0 + 1, buf1, sem1)
        _wait(buf0, sem0)
        _row_lse(buf0, r0)

        @pl.when(gidx < _PW // 2 - 1)
        def _prefetch():
            _start(r0 + 2, buf0, sem0)

        _wait(buf1, sem1)
        _row_lse(buf1, r0 + 1)
        return carry

    lax.fori_loop(0, _PW // 2, row_pair, 0)
    pltpu.sync_copy(m_buf, m_out.at[wid])
    pltpu.sync_copy(s_buf, s_out.at[wid])


def _combine_body(mt_ref, st_ref, ms_ref, ss_ref, lab_ref, g_ref,
                  cp_ref, den_ref, out_ref):
    mt = mt_ref[:, :]
    st = st_ref[:, :]
    m3 = ms_ref[...]
    s3 = ss_ref[...]
    ms = jnp.max(m3, axis=2)
    ss = jnp.sum(s3 * jnp.exp(m3 - ms[:, :, None]), axis=2)
    big = jnp.maximum(mt, ms)
    tot = st * jnp.exp(mt - big) + ss * jnp.exp(ms - big)
    lse = big + jnp.log(tot)
    valid = lab_ref[:, :] != _IGNORE
    num = jnp.sum(jnp.where(valid, lse, 0.0)) - jnp.sum(g_ref[:, :])
    cnt = jnp.sum(valid.astype(jnp.float32))
    aux = _NUM_EXPERTS * cp_ref[0, 0] / (den_ref[0, 0] * den_ref[0, 0])
    out_ref[:, :] = (num / cnt + _AUX_COEF * aux).reshape(1, 1)


def kernel(logits, labels, router_logits):
    n = logits.shape[1]
    v = logits.shape[-1]
    logits2 = logits.reshape(n, v)
    lab_flat = labels.reshape(-1)
    shift_lab = jnp.concatenate(
        [lab_flat[1:], jnp.full((1,), _IGNORE, jnp.int32)])

    sc_call = functools.partial(
        pl.kernel,
        mesh=plsc.VectorSubcoreMesh(core_axis_name="c", subcore_axis_name="s"),
        out_type=[
            jax.ShapeDtypeStruct((_NW, _SCL), jnp.float32),
            jax.ShapeDtypeStruct((_NW, _PW * _SCL), jnp.float32),
            jax.ShapeDtypeStruct((_NW, _PW * _SCL), jnp.float32),
        ],
        scratch_types=[
            pltpu.VMEM((_PW,), jnp.int32),
            pltpu.VMEM((_PW,), jnp.int32),
            pltpu.VMEM((_PW,), jnp.float32),
            pltpu.VMEM((_SCL,), jnp.float32),
            pltpu.VMEM((_VSC,), jnp.float32),
            pltpu.VMEM((_VSC,), jnp.float32),
            pltpu.VMEM((_PW * _SCL,), jnp.float32),
            pltpu.VMEM((_PW * _SCL,), jnp.float32),
            pltpu.SemaphoreType.DMA,
            pltpu.SemaphoreType.DMA,
            pltpu.SemaphoreType.DMA,
        ],
    )(_sc_body)
    g_partials, m_sc, s_sc = sc_call(logits.reshape(-1), shift_lab)

    lab2 = lab_flat.reshape(1, n)
    router_t = router_logits.transpose(0, 2, 1)
    m_tc, s_tc, cp, den = pl.pallas_call(
        _fused_body,
        grid=(n // _R,),
        in_specs=[
            pl.BlockSpec((_R, _VTC), lambda i: (i, 0)),
            pl.BlockSpec((_LPB, _E, _N), lambda i: (i, 0, 0)),
            pl.BlockSpec((1, _N), lambda i: (0, 0)),
        ],
        out_specs=[
            pl.BlockSpec((_R, 1), lambda i: (i, 0)),
            pl.BlockSpec((_R, 1), lambda i: (i, 0)),
            pl.BlockSpec((1, 1), lambda i: (0, 0)),
            pl.BlockSpec((1, 1), lambda i: (0, 0)),
        ],
        out_shape=[
            jax.ShapeDtypeStruct((n, 1), jnp.float32),
            jax.ShapeDtypeStruct((n, 1), jnp.float32),
            jax.ShapeDtypeStruct((1, 1), jnp.float32),
            jax.ShapeDtypeStruct((1, 1), jnp.float32),
        ],
        scratch_shapes=[
            pltpu.VMEM((_E, 128), jnp.float32),
            pltpu.VMEM((_E, 128), jnp.float32),
            pltpu.VMEM((1, 128), jnp.float32),
        ],
        compiler_params=pltpu.CompilerParams(
            dimension_semantics=("arbitrary",)),
    )(logits2, router_t, lab2)

    shp = (n // 128, 128)
    shp3 = (n // 128, 128, _SCL)
    loss = pl.pallas_call(
        _combine_body,
        grid=(1,),
        in_specs=[
            pl.BlockSpec(shp, lambda i: (0, 0)),
            pl.BlockSpec(shp, lambda i: (0, 0)),
            pl.BlockSpec(shp3, lambda i: (0, 0, 0)),
            pl.BlockSpec(shp3, lambda i: (0, 0, 0)),
            pl.BlockSpec(shp, lambda i: (0, 0)),
            pl.BlockSpec((_NW, _SCL), lambda i: (0, 0)),
            pl.BlockSpec((1, 1), lambda i: (0, 0)),
            pl.BlockSpec((1, 1), lambda i: (0, 0)),
        ],
        out_specs=pl.BlockSpec((1, 1), lambda i: (0, 0)),
        out_shape=jax.ShapeDtypeStruct((1, 1), jnp.float32),
    )(m_tc.reshape(shp), s_tc.reshape(shp),
      m_sc.reshape(shp3), s_sc.reshape(shp3),
      shift_lab.reshape(shp), g_partials, cp, den)
    return loss[0, 0]
